# baseline (speedup 1.0000x reference)
import functools

import jax
import jax.numpy as jnp
from jax import lax
from jax.experimental import pallas as pl
from jax.experimental.pallas import tpu as pltpu
from jax.experimental.pallas import tpu_sc as plsc

N_NODES = 10000
N_EDGES = 320000
D = 128
H = 512
QW = 2
CHUNK = 128
N_CHUNKS = N_EDGES // CHUNK
NC, NS = 2, 16
NW = NC * NS
ROWS_PER_TILE = N_NODES // NS
EPS = 0.1


STRIPE = 624
REM0 = NS * STRIPE
REM = N_NODES - REM0
NQ = 20480
QSTRIPE = NQ // NS
NLANE = 16


NBUF = 2
K_MAIN = N_CHUNKS // NW
N_REM = N_CHUNKS - K_MAIN * NW


def _sc_body(ei_hbm, pred_hbm, zp_hbm, zq_hbm,
             p_out, q_out,
             h0, h1, t0, t1, v0, v1,
             fidx_v, ones_v, shared_p, shared_q, sem0, sem1):
    c = lax.axis_index("c")
    s = lax.axis_index("s")
    wid = c * NS + s
    hidx = [h0, h1]
    tidx = [t0, t1]
    vals = [v0, v1]
    sems = [sem0, sem1]

    def fetch(chunk, b):
        e0 = pl.multiple_of(chunk * CHUNK, CHUNK)
        pltpu.async_copy(pred_hbm.at[pl.ds(e0, CHUNK)], vals[b], sems[b])
        pltpu.async_copy(ei_hbm.at[0, pl.ds(e0, CHUNK)], hidx[b], sems[b])
        pltpu.async_copy(ei_hbm.at[1, pl.ds(e0, CHUNK)], tidx[b], sems[b])

    def drain(chunk, b):
        e0 = pl.multiple_of(chunk * CHUNK, CHUNK)
        pltpu.make_async_copy(pred_hbm.at[pl.ds(e0, CHUNK)], vals[b],
                              sems[b]).wait()
        pltpu.make_async_copy(ei_hbm.at[0, pl.ds(e0, CHUNK)], hidx[b],
                              sems[b]).wait()
        pltpu.make_async_copy(ei_hbm.at[1, pl.ds(e0, CHUNK)], tidx[b],
                              sems[b]).wait()

    def scatter(hv, tv, vv):
        pltpu.sync_copy(vv, shared_p.at[hv], add=True)
        pltpu.sync_copy(vv, shared_p.at[tv], add=True)
        for i in range(CHUNK // NLANE):
            sl = pl.ds(i * NLANE, NLANE)
            fidx_v[sl] = hv[sl] * 2
        pltpu.sync_copy(ones_v, shared_q.at[fidx_v], add=True)
        for i in range(CHUNK // NLANE):
            sl = pl.ds(i * NLANE, NLANE)
            fidx_v[sl] = tv[sl] * 2 + 1
        pltpu.sync_copy(ones_v, shared_q.at[fidx_v], add=True)

    for b in range(NBUF):
        fetch(b * NW + wid, b)

    r0 = pl.multiple_of(s * STRIPE, 8)
    q0 = pl.multiple_of(s * QSTRIPE, 128)
    pltpu.sync_copy(zp_hbm.at[pl.ds(r0, STRIPE)],
                    shared_p.at[pl.ds(r0, STRIPE)])
    pltpu.sync_copy(zq_hbm.at[pl.ds(q0, QSTRIPE)],
                    shared_q.at[pl.ds(q0, QSTRIPE)])

    @pl.when(s == NS - 1)
    def _():
        pltpu.sync_copy(zp_hbm.at[pl.ds(REM0, REM)],
                        shared_p.at[pl.ds(REM0, REM)])

    one16 = jnp.ones((NLANE,), jnp.float32)
    for i in range(CHUNK // NLANE):
        ones_v[pl.ds(i * NLANE, NLANE)] = one16
    plsc.subcore_barrier()

    def body(g, carry):
        for b in range(NBUF):
            k = g * NBUF + b
            chunk = k * NW + wid
            drain(chunk, b)
            scatter(hidx[b], tidx[b], vals[b])

            @pl.when(k + NBUF < K_MAIN)
            def _():
                fetch(chunk + NBUF * NW, b)

        return carry

    lax.fori_loop(0, K_MAIN // NBUF, body, 0)

    @pl.when(wid < N_REM)
    def _():
        chunk = K_MAIN * NW + wid
        e0 = pl.multiple_of(chunk * CHUNK, CHUNK)
        pltpu.sync_copy(ei_hbm.at[0, pl.ds(e0, CHUNK)], h0)
        pltpu.sync_copy(ei_hbm.at[1, pl.ds(e0, CHUNK)], t0)
        pltpu.sync_copy(pred_hbm.at[pl.ds(e0, CHUNK)], v0)
        scatter(h0, t0, v0)

    plsc.subcore_barrier()
    pltpu.sync_copy(shared_p.at[pl.ds(r0, STRIPE)],
                    p_out.at[c, pl.ds(r0, STRIPE)])
    pltpu.sync_copy(shared_q.at[pl.ds(q0, QSTRIPE)],
                    q_out.at[c, 0, pl.ds(q0, QSTRIPE)])

    @pl.when(s == NS - 1)
    def _():
        pltpu.sync_copy(shared_p.at[pl.ds(REM0, REM)],
                        p_out.at[c, pl.ds(REM0, REM)])


@functools.lru_cache(maxsize=None)
def _get_sc_scatter():
    return pl.kernel(
        _sc_body,
        out_type=[jax.ShapeDtypeStruct((NC, N_NODES, D), jnp.float32),
                  jax.ShapeDtypeStruct((NC, 1, NQ), jnp.float32)],
        mesh=plsc.VectorSubcoreMesh(core_axis_name="c", subcore_axis_name="s",
                                    num_cores=NC, num_subcores=NS),
        scratch_types=(
            [pltpu.VMEM((CHUNK,), jnp.int32) for _ in range(2 * NBUF)]
            + [pltpu.VMEM((CHUNK, D), jnp.float32) for _ in range(NBUF)]
            + [
                pltpu.VMEM((CHUNK,), jnp.int32),
                pltpu.VMEM((CHUNK,), jnp.float32),
                pltpu.VMEM_SHARED((N_NODES, D), jnp.float32),
                pltpu.VMEM_SHARED((NQ,), jnp.float32),
            ]
            + [pltpu.SemaphoreType.DMA for _ in range(NBUF)]
        ),
    )


def _tc_body(p_ref, q_ref, x_ref, w1, b1, w2, b2,
             m1, c1, m2, c2, o_ref):
    P = p_ref[0] + p_ref[1]
    q = q_ref[0] + q_ref[1]
    degh = q[:, 0:1]
    degt = q[:, 1:2]
    dsum = degh + degt
    ddiff = degh - degt
    x = x_ref[...]
    w1a = w1[0:D]
    w1b = w1[D:2 * D]
    w1cd = w1[2 * D:2 * D + 2]
    h = jnp.dot(dsum * x, w1a, preferred_element_type=jnp.float32)
    h = h + jnp.dot(P, w1b, preferred_element_type=jnp.float32)
    h = h + ddiff * w1cd[0:1] + dsum * w1cd[1:2] + b1[...]
    h = jnp.maximum(h, 0.0)
    enc = jnp.dot(h, w2[...], preferred_element_type=jnp.float32) + b2[...]
    t = enc + EPS * x
    h2 = jnp.maximum(
        jnp.dot(t, m1[...], preferred_element_type=jnp.float32) + c1[...], 0.0)
    o_ref[...] = jnp.dot(h2, m2[...], preferred_element_type=jnp.float32) + c2[...]


_TC_BLOCK = 2000


def _tc_dense(P2, Q2, x, W1, b1, W2, b2, M1, c1, M2, c2):
    n_blocks = N_NODES // _TC_BLOCK
    full = lambda shape: pl.BlockSpec(shape, lambda i: (0,) * len(shape))
    return pl.pallas_call(
        _tc_body,
        grid=(n_blocks,),
        in_specs=[
            pl.BlockSpec((NC, _TC_BLOCK, D), lambda i: (0, i, 0)),
            pl.BlockSpec((NC, _TC_BLOCK, 2), lambda i: (0, i, 0)),
            pl.BlockSpec((_TC_BLOCK, D), lambda i: (i, 0)),
            full((2 * D + 2, H)), full((H,)),
            full((H, D)), full((D,)),
            full((D, H)), full((H,)), full((H, D)), full((D,)),
        ],
        out_specs=pl.BlockSpec((_TC_BLOCK, D), lambda i: (i, 0)),
        out_shape=jax.ShapeDtypeStruct((N_NODES, D), jnp.float32),
    )(P2, Q2, x, W1, b1, W2, b2, M1, c1, M2, c2)


@jax.jit
def kernel(x, edge_index, pred_emb, sign, W1, b1, W2, b2, M1, c1, M2, c2):
    del sign
    zp = jnp.zeros((N_NODES, D), jnp.float32)
    zq = jnp.zeros((NQ,), jnp.float32)
    P2, Q2 = _get_sc_scatter()(edge_index, pred_emb, zp, zq)
    return _tc_dense(
        P2, Q2.reshape(NC, NQ // 2, 2), x,
        W1, b1, W2, b2, M1, c1, M2, c2)


# speedup vs baseline: 1.0235x; 1.0235x over previous
import functools

import jax
import jax.numpy as jnp
from jax import lax
from jax.experimental import pallas as pl
from jax.experimental.pallas import tpu as pltpu
from jax.experimental.pallas import tpu_sc as plsc

N_NODES = 10000
N_EDGES = 320000
D = 128
H = 512
QW = 2
CHUNK = 128
N_CHUNKS = N_EDGES // CHUNK
NC, NS = 2, 16
NW = NC * NS
ROWS_PER_TILE = N_NODES // NS
EPS = 0.1


STRIPE = 624
REM0 = NS * STRIPE
REM = N_NODES - REM0
NQ = 20480
QSTRIPE = NQ // NS
NLANE = 16


NBUF = 2
K_MAIN = N_CHUNKS // NW
N_REM = N_CHUNKS - K_MAIN * NW


def _sc_body(ei_hbm, pred_hbm, zp_hbm, zq_hbm,
             p_out, q_out,
             h0, h1, t0, t1, v0, v1,
             fidx_v, ones_v, shared_p, shared_q, sem0, sem1):
    c = lax.axis_index("c")
    s = lax.axis_index("s")
    wid = c * NS + s
    hidx = [h0, h1]
    tidx = [t0, t1]
    vals = [v0, v1]
    sems = [sem0, sem1]

    def fetch(chunk, b):
        e0 = pl.multiple_of(chunk * CHUNK, CHUNK)
        pltpu.async_copy(pred_hbm.at[pl.ds(e0, CHUNK)], vals[b], sems[b])
        pltpu.async_copy(ei_hbm.at[0, pl.ds(e0, CHUNK)], hidx[b], sems[b])
        pltpu.async_copy(ei_hbm.at[1, pl.ds(e0, CHUNK)], tidx[b], sems[b])

    def drain(chunk, b):
        e0 = pl.multiple_of(chunk * CHUNK, CHUNK)
        pltpu.make_async_copy(pred_hbm.at[pl.ds(e0, CHUNK)], vals[b],
                              sems[b]).wait()
        pltpu.make_async_copy(ei_hbm.at[0, pl.ds(e0, CHUNK)], hidx[b],
                              sems[b]).wait()
        pltpu.make_async_copy(ei_hbm.at[1, pl.ds(e0, CHUNK)], tidx[b],
                              sems[b]).wait()

    def scatter(hv, tv, vv):
        pltpu.sync_copy(vv, shared_p.at[hv], add=True)
        pltpu.sync_copy(vv, shared_p.at[tv], add=True)
        for i in range(CHUNK // NLANE):
            sl = pl.ds(i * NLANE, NLANE)
            fidx_v[sl] = hv[sl] * 2
        pltpu.sync_copy(ones_v, shared_q.at[fidx_v], add=True)
        for i in range(CHUNK // NLANE):
            sl = pl.ds(i * NLANE, NLANE)
            fidx_v[sl] = tv[sl] * 2 + 1
        pltpu.sync_copy(ones_v, shared_q.at[fidx_v], add=True)

    for b in range(NBUF):
        fetch(b * NW + wid, b)

    r0 = pl.multiple_of(s * STRIPE, 8)
    q0 = pl.multiple_of(s * QSTRIPE, 128)
    pltpu.sync_copy(zp_hbm.at[pl.ds(r0, STRIPE)],
                    shared_p.at[pl.ds(r0, STRIPE)])
    pltpu.sync_copy(zq_hbm.at[pl.ds(q0, QSTRIPE)],
                    shared_q.at[pl.ds(q0, QSTRIPE)])

    @pl.when(s == NS - 1)
    def _():
        pltpu.sync_copy(zp_hbm.at[pl.ds(REM0, REM)],
                        shared_p.at[pl.ds(REM0, REM)])

    one16 = jnp.ones((NLANE,), jnp.float32)
    for i in range(CHUNK // NLANE):
        ones_v[pl.ds(i * NLANE, NLANE)] = one16
    plsc.subcore_barrier()

    def body(g, carry):
        for b in range(NBUF):
            k = g * NBUF + b
            chunk = k * NW + wid
            drain(chunk, b)
            scatter(hidx[b], tidx[b], vals[b])

            @pl.when(k + NBUF < K_MAIN)
            def _():
                fetch(chunk + NBUF * NW, b)

        return carry

    lax.fori_loop(0, K_MAIN // NBUF, body, 0)

    @pl.when(wid < N_REM)
    def _():
        chunk = K_MAIN * NW + wid
        e0 = pl.multiple_of(chunk * CHUNK, CHUNK)
        pltpu.sync_copy(ei_hbm.at[0, pl.ds(e0, CHUNK)], h0)
        pltpu.sync_copy(ei_hbm.at[1, pl.ds(e0, CHUNK)], t0)
        pltpu.sync_copy(pred_hbm.at[pl.ds(e0, CHUNK)], v0)
        scatter(h0, t0, v0)

    plsc.subcore_barrier()
    pltpu.sync_copy(shared_p.at[pl.ds(r0, STRIPE)],
                    p_out.at[c, pl.ds(r0, STRIPE)])
    pltpu.sync_copy(shared_q.at[pl.ds(q0, QSTRIPE)],
                    q_out.at[c, 0, pl.ds(q0, QSTRIPE)])

    @pl.when(s == NS - 1)
    def _():
        pltpu.sync_copy(shared_p.at[pl.ds(REM0, REM)],
                        p_out.at[c, pl.ds(REM0, REM)])


@functools.lru_cache(maxsize=None)
def _get_sc_scatter():
    return pl.kernel(
        _sc_body,
        out_type=[jax.ShapeDtypeStruct((NC, N_NODES, D), jnp.float32),
                  jax.ShapeDtypeStruct((NC, 1, NQ), jnp.float32)],
        mesh=plsc.VectorSubcoreMesh(core_axis_name="c", subcore_axis_name="s",
                                    num_cores=NC, num_subcores=NS),
        scratch_types=(
            [pltpu.VMEM((CHUNK,), jnp.int32) for _ in range(2 * NBUF)]
            + [pltpu.VMEM((CHUNK, D), jnp.float32) for _ in range(NBUF)]
            + [
                pltpu.VMEM((CHUNK,), jnp.int32),
                pltpu.VMEM((CHUNK,), jnp.float32),
                pltpu.VMEM_SHARED((N_NODES, D), jnp.float32),
                pltpu.VMEM_SHARED((NQ,), jnp.float32),
            ]
            + [pltpu.SemaphoreType.DMA for _ in range(NBUF)]
        ),
    )


def _tc_body(p_ref, q_ref, x_ref, w1, b1, w2, b2,
             m1, c1, m2, c2, o_ref):
    P = p_ref[0] + p_ref[1]
    q = q_ref[0] + q_ref[1]
    degh = q[:, 0:1]
    degt = q[:, 1:2]
    dsum = degh + degt
    ddiff = degh - degt
    x = x_ref[...]
    w1a = w1[0:D]
    w1b = w1[D:2 * D]
    w1cd = w1[2 * D:2 * D + 2]
    h = jnp.dot(dsum * x, w1a, preferred_element_type=jnp.float32)
    h = h + jnp.dot(P, w1b, preferred_element_type=jnp.float32)
    h = h + ddiff * w1cd[0:1] + dsum * w1cd[1:2] + b1[...]
    h = jnp.maximum(h, 0.0)
    enc = jnp.dot(h, w2[...], preferred_element_type=jnp.float32) + b2[...]
    t = enc + EPS * x
    h2 = jnp.maximum(
        jnp.dot(t, m1[...], preferred_element_type=jnp.float32) + c1[...], 0.0)
    o_ref[...] = jnp.dot(h2, m2[...], preferred_element_type=jnp.float32) + c2[...]


_TC_BLOCK = 1000


def _tc_dense(P2, Q2, x, W1, b1, W2, b2, M1, c1, M2, c2):
    n_blocks = N_NODES // _TC_BLOCK
    full = lambda shape: pl.BlockSpec(shape, lambda i: (0,) * len(shape))
    return pl.pallas_call(
        _tc_body,
        grid=(n_blocks,),
        in_specs=[
            pl.BlockSpec((NC, _TC_BLOCK, D), lambda i: (0, i, 0)),
            pl.BlockSpec((NC, _TC_BLOCK, 2), lambda i: (0, i, 0)),
            pl.BlockSpec((_TC_BLOCK, D), lambda i: (i, 0)),
            full((2 * D + 2, H)), full((H,)),
            full((H, D)), full((D,)),
            full((D, H)), full((H,)), full((H, D)), full((D,)),
        ],
        out_specs=pl.BlockSpec((_TC_BLOCK, D), lambda i: (i, 0)),
        out_shape=jax.ShapeDtypeStruct((N_NODES, D), jnp.float32),
    )(P2, Q2, x, W1, b1, W2, b2, M1, c1, M2, c2)


@jax.jit
def kernel(x, edge_index, pred_emb, sign, W1, b1, W2, b2, M1, c1, M2, c2):
    del sign
    zp = jnp.zeros((N_NODES, D), jnp.float32)
    zq = jnp.zeros((NQ,), jnp.float32)
    P2, Q2 = _get_sc_scatter()(edge_index, pred_emb, zp, zq)
    return _tc_dense(
        P2, Q2[:, 0, :2 * N_NODES].reshape(NC, N_NODES, 2), x,
        W1, b1, W2, b2, M1, c1, M2, c2)
